# baseline (speedup 1.0000x reference)
import jax
import jax.numpy as jnp
from jax.experimental import pallas as pl

_TF = 512


def _mlp_kernel(x_ref, wg_ref, wu_ref, wd_ref, o_ref):
    j = pl.program_id(0)
    x = x_ref[...].astype(jnp.bfloat16)
    g = jnp.dot(x, wg_ref[...].astype(jnp.bfloat16),
                preferred_element_type=jnp.float32)
    u = jnp.dot(x, wu_ref[...].astype(jnp.bfloat16),
                preferred_element_type=jnp.float32)
    a = (g * jax.nn.sigmoid(g)) * u
    p = jnp.dot(a.astype(jnp.bfloat16), wd_ref[...].astype(jnp.bfloat16),
                preferred_element_type=jnp.float32)

    @pl.when(j == 0)
    def _init():
        o_ref[...] = p

    @pl.when(j > 0)
    def _acc():
        o_ref[...] += p


@jax.jit
def kernel(hidden_states, router_w, w_gate_proj, w_up_proj, w_down_proj):
    b, s, d = hidden_states.shape
    n = b * s
    ff = w_gate_proj.shape[1]
    x = hidden_states.reshape(n, d)
    out = pl.pallas_call(
        _mlp_kernel,
        grid=(ff // _TF,),
        in_specs=[
            pl.BlockSpec((n, d), lambda j: (0, 0)),
            pl.BlockSpec((d, _TF), lambda j: (0, j)),
            pl.BlockSpec((d, _TF), lambda j: (0, j)),
            pl.BlockSpec((_TF, d), lambda j: (j, 0)),
        ],
        out_specs=pl.BlockSpec((n, d), lambda j: (0, 0)),
        out_shape=jax.ShapeDtypeStruct((n, d), jnp.float32),
    )(x, w_gate_proj, w_up_proj, w_down_proj)
    return out.reshape(b, s, d)
